# baseline (speedup 1.0000x reference)
import functools

import jax
import jax.numpy as jnp
from jax.experimental import pallas as pl
from jax.experimental.pallas import tpu as pltpu

N = 4096
IN_FEAT = 128
BLK = 512
GRID = N // BLK


def _dot(a, b):
    return jax.lax.dot_general(
        a, b, (((1,), (0,)), ((), ())),
        preferred_element_type=jnp.float32,
    )


def _fused_kernel(adj_ref, b_ref, w_enc_ref, x_latent_ref, x_recon_ref, ab_acc):
    i = pl.program_id(0)
    ab = _dot(adj_ref[...].astype(jnp.bfloat16), b_ref[...])
    ab_acc[pl.ds(i * BLK, BLK), :] = ab
    x_latent_ref[...] = _dot(
        ab[:, :IN_FEAT].astype(jnp.bfloat16), w_enc_ref[...])

    @pl.when(i == GRID - 1)
    def _():
        a = ab_acc[:, :IN_FEAT].astype(jnp.bfloat16)
        y = ab_acc[:, IN_FEAT:].astype(jnp.bfloat16)
        m = _dot(w_enc_ref[...], y)
        x_recon_ref[...] = _dot(a, m.astype(jnp.bfloat16))


@jax.jit
def _run(feat, adj, W_enc, W_dec):
    b = jnp.concatenate([feat, W_dec], axis=1).astype(jnp.bfloat16)
    x_latent, x_recon = pl.pallas_call(
        _fused_kernel,
        grid=(GRID,),
        in_specs=[
            pl.BlockSpec((BLK, N), lambda i: (i, 0)),
            pl.BlockSpec((N, 2 * IN_FEAT), lambda i: (0, 0)),
            pl.BlockSpec((IN_FEAT, N), lambda i: (0, 0)),
        ],
        out_specs=[
            pl.BlockSpec((BLK, N), lambda i: (i, 0)),
            pl.BlockSpec((N, IN_FEAT), lambda i: (0, 0)),
        ],
        out_shape=[
            jax.ShapeDtypeStruct((N, N), jnp.float32),
            jax.ShapeDtypeStruct((N, IN_FEAT), jnp.float32),
        ],
        scratch_shapes=[pltpu.VMEM((N, 2 * IN_FEAT), jnp.float32)],
    )(adj, b, W_enc.astype(jnp.bfloat16))
    return x_latent, x_recon


def kernel(feat, adj, W_enc, W_dec):
    return _run(feat, adj, W_enc, W_dec)


# speedup vs baseline: 1.0053x; 1.0053x over previous
import functools

import jax
import jax.numpy as jnp
from jax.experimental import pallas as pl
from jax.experimental.pallas import tpu as pltpu

N = 4096
IN_FEAT = 128
BLK1 = 1024
BLK2 = 512


def _dot(a, b):
    return jax.lax.dot_general(
        a, b, (((1,), (0,)), ((), ())),
        preferred_element_type=jnp.float32,
    )


def _pass1(adj_ref, b_ref, ab_ref):
    ab_ref[...] = _dot(adj_ref[...], b_ref[...])


def _pass2(ab_ref, w_enc_ref, x_latent_ref, x_recon_ref, m_ref):
    @pl.when(pl.program_id(0) == 0)
    def _():
        m_ref[...] = _dot(w_enc_ref[...], ab_ref[:, IN_FEAT:])

    a_blk = ab_ref[pl.ds(pl.program_id(0) * BLK2, BLK2), :IN_FEAT]
    x_latent_ref[...] = _dot(a_blk, w_enc_ref[...])
    x_recon_ref[...] = _dot(a_blk, m_ref[...])


@jax.jit
def _run(feat, adj, W_enc, W_dec):
    b = jnp.concatenate([feat, W_dec], axis=1)
    ab = pl.pallas_call(
        _pass1,
        grid=(N // BLK1,),
        in_specs=[
            pl.BlockSpec((BLK1, N), lambda i: (i, 0)),
            pl.BlockSpec((N, 2 * IN_FEAT), lambda i: (0, 0)),
        ],
        out_specs=pl.BlockSpec((BLK1, 2 * IN_FEAT), lambda i: (i, 0)),
        out_shape=jax.ShapeDtypeStruct((N, 2 * IN_FEAT), jnp.float32),
    )(adj, b)
    x_latent, x_recon = pl.pallas_call(
        _pass2,
        grid=(N // BLK2,),
        in_specs=[
            pl.BlockSpec((N, 2 * IN_FEAT), lambda i: (0, 0)),
            pl.BlockSpec((IN_FEAT, N), lambda i: (0, 0)),
        ],
        out_specs=[
            pl.BlockSpec((BLK2, N), lambda i: (i, 0)),
            pl.BlockSpec((BLK2, IN_FEAT), lambda i: (i, 0)),
        ],
        out_shape=[
            jax.ShapeDtypeStruct((N, N), jnp.float32),
            jax.ShapeDtypeStruct((N, IN_FEAT), jnp.float32),
        ],
        scratch_shapes=[pltpu.VMEM((IN_FEAT, IN_FEAT), jnp.float32)],
    )(ab, W_enc)
    return x_latent, x_recon


def kernel(feat, adj, W_enc, W_dec):
    return _run(feat, adj, W_enc, W_dec)
